# baseline (speedup 1.0000x reference)
import functools

import jax
import jax.numpy as jnp
from jax import lax
from jax.experimental import pallas as pl
from jax.experimental.pallas import tpu as pltpu
from jax.experimental.pallas import tpu_sc as plsc

N = 10000
D = 128
E = 320000
EPS = 1e-05

NC = 2
NS = 16
NW = NC * NS
CHUNK = 128
CPW = 88
E_PAD = NW * CPW * CHUNK
E_TOT = E + N
CPC = NW * CPW // NS
RNG = 1792
NRG = 6
RPT = RNG // NS
ACC_R = RNG + 8
ZR = 112

_mesh = plsc.VectorSubcoreMesh(core_axis_name="c", subcore_axis_name="s")


@functools.partial(
    pl.kernel,
    out_type=jax.ShapeDtypeStruct((NRG, RNG, D), jnp.float32),
    mesh=_mesh,
    scratch_types=[
        pltpu.VMEM((CPC, CHUNK), jnp.int32),
        pltpu.VMEM((CPC + 2, CHUNK), jnp.int32),
        pltpu.VMEM((CPC, CHUNK), jnp.int32),
        pltpu.VMEM((CHUNK, D), jnp.float32),
        pltpu.VMEM((CHUNK, D), jnp.float32),
        pltpu.VMEM((ZR, D), jnp.float32),
        pltpu.VMEM_SHARED((ACC_R, D), jnp.float32),
        pltpu.SemaphoreType.DMA,
        pltpu.SemaphoreType.DMA,
    ],
)
def _scatter(g_hbm, pk_hbm, out_hbm, pk_v, src_v, dst_v,
             rows0, rows1, zbuf, acc, sem0, sem1):
    c = lax.axis_index("c")
    s = lax.axis_index("s")

    def fill_zero(i, _):
        zbuf[i // 8, pl.ds((i % 8) * 16, 16)] = jnp.zeros((16,), jnp.float32)
        return 0
    lax.fori_loop(0, ZR * 8, fill_zero, 0)

    pltpu.sync_copy(pk_hbm.at[pl.ds(s * CPC, CPC)], pk_v)

    def unpack_src(i, _):
        v = pk_v[i // 8, pl.ds((i % 8) * 16, 16)]
        src_v[i // 8, pl.ds((i % 8) * 16, 16)] = lax.shift_right_logical(v, 14)
        return 0
    lax.fori_loop(0, CPC * 8, unpack_src, 0)

    for i in range(8):
        src_v[CPC, pl.ds(i * 16, 16)] = jnp.zeros((16,), jnp.int32)
        src_v[CPC + 1, pl.ds(i * 16, 16)] = jnp.zeros((16,), jnp.int32)

    for p in range(NRG // 2):
        r = c * (NRG // 2) + p
        base = r * RNG

        def unpack_dst(i, _):
            v = pk_v[i // 8, pl.ds((i % 8) * 16, 16)]
            d = (v & 16383) - base
            ok = (d >= 0) & (d < RNG)
            dst_v[i // 8, pl.ds((i % 8) * 16, 16)] = jnp.where(ok, d, RNG)
            return 0
        lax.fori_loop(0, CPC * 8, unpack_dst, 0)

        def zero_acc(k, _):
            pltpu.sync_copy(zbuf, acc.at[pl.ds(s * RPT + k * ZR, ZR)])
            return 0
        lax.fori_loop(0, RPT // ZR, zero_acc, 0)
        plsc.subcore_barrier()

        pltpu.async_copy(g_hbm.at[src_v.at[0]], rows0, sem0)
        pltpu.async_copy(g_hbm.at[src_v.at[1]], rows1, sem1)

        def body(g, _):
            for b, (rows, sem) in enumerate(((rows0, sem0), (rows1, sem1))):
                j = 2 * g + b
                pltpu.make_async_copy(g_hbm.at[src_v.at[j]], rows, sem).wait()
                pltpu.sync_copy(rows, acc.at[dst_v.at[j]], add=True)
                pltpu.async_copy(g_hbm.at[src_v.at[j + 2]], rows, sem)
            return 0
        lax.fori_loop(0, CPC // 2, body, 0)

        pltpu.make_async_copy(g_hbm.at[src_v.at[CPC]], rows0, sem0).wait()
        pltpu.make_async_copy(g_hbm.at[src_v.at[CPC + 1]], rows1, sem1).wait()

        plsc.subcore_barrier()
        pltpu.sync_copy(
            acc.at[pl.ds(s * RPT, RPT)],
            out_hbm.at[r, pl.ds(s * RPT, RPT)],
        )
        plsc.subcore_barrier()


def _dinv_of(degc):
    return jnp.where(degc > 0.0, lax.rsqrt(degc), 0.0)


def _assemble(sp):
    return sp.reshape(NRG * RNG, D)[:N]


def _pack_body(e0_ref, e1_ref, o_ref):
    head = (e0_ref[...] << 14) | e1_ref[...]
    rows = NW * CPW - E // CHUNK
    n = (lax.broadcasted_iota(jnp.int32, (rows, CHUNK), 0) * CHUNK
         + lax.broadcasted_iota(jnp.int32, (rows, CHUNK), 1))
    tail = jnp.where(n < N, n * ((1 << 14) + 1), N)
    o_ref[...] = jnp.concatenate([head, tail], axis=0)


def _degcol_body(dsp_ref, o_ref):
    o_ref[...] = _assemble(dsp_ref[...])[:, 0:1]


def _first_body(x_ref, w_ref, o_ref):
    h = jnp.clip(x_ref[...], -10.0, 10.0)
    o_ref[...] = jnp.dot(h, w_ref[...], preferred_element_type=jnp.float32)


def _scale_body(m_ref, degc_ref, o_ref):
    o_ref[...] = m_ref[...] * _dinv_of(degc_ref[...])


def _mid_body(sp_ref, degc_ref, b_ref, g_ref, bt_ref, w_ref, og_ref, oh_ref):
    dinv = _dinv_of(degc_ref[...])
    t = _assemble(sp_ref[...]) * dinv + b_ref[...]
    mu = jnp.mean(t, axis=0, keepdims=True)
    cen = t - mu
    var = jnp.mean(cen * cen, axis=0, keepdims=True)
    hbn = cen * (g_ref[...] * lax.rsqrt(var + EPS)) + bt_ref[...]
    oh_ref[...] = hbn
    h = jnp.maximum(hbn, 0.0)
    og_ref[...] = jnp.dot(h, w_ref[...], preferred_element_type=jnp.float32) * dinv


_nd = jax.ShapeDtypeStruct((N, D), jnp.float32)
_pack = pl.pallas_call(
    _pack_body, out_shape=jax.ShapeDtypeStruct((NW * CPW, CHUNK), jnp.int32))
_degcol = pl.pallas_call(
    _degcol_body, out_shape=jax.ShapeDtypeStruct((N, 1), jnp.float32))
_first = pl.pallas_call(_first_body, out_shape=_nd)
_scale = pl.pallas_call(_scale_body, out_shape=_nd)
_mid = pl.pallas_call(_mid_body, out_shape=(_nd, _nd))


def kernel(x, edge_index, W1, b1, g1, bt1, W2, b2, g2, bt2, W3, b3, g3, bt3):
    pk2d = _pack(edge_index[0].reshape(E // CHUNK, CHUNK),
                 edge_index[1].reshape(E // CHUNK, CHUNK))

    bs = jnp.stack([b1, b2, b3])[:, None, :]
    gs = jnp.stack([g1, g2, g3])[:, None, :]
    bts = jnp.stack([bt1, bt2, bt3])[:, None, :]
    ws = jnp.stack([W2, W3, jnp.eye(D, dtype=jnp.float32)])

    degc = _degcol(_scatter(jnp.ones((N, D), jnp.float32), pk2d))
    m1 = _first(x, W1)
    g0 = _scale(m1, degc)

    def step(g, params):
        b, gam, bt, w = params
        sp = _scatter(g, pk2d)
        g_next, hbn = _mid(sp, degc, b, gam, bt, w)
        return g_next, hbn

    _, hbns = lax.scan(step, g0, (bs, gs, bts, ws))
    return hbns[-1]


# speedup vs baseline: 1.6827x; 1.6827x over previous
import functools

import jax
import jax.numpy as jnp
from jax import lax
from jax.experimental import pallas as pl
from jax.experimental.pallas import tpu as pltpu
from jax.experimental.pallas import tpu_sc as plsc

N = 10000
D = 128
E = 320000
EPS = 1e-05

NC = 2
NS = 16
NW = NC * NS
CHUNK = 128
CPW = 88
E_PAD = NW * CPW * CHUNK
E_TOT = E + N
CPC = NW * CPW // NS
RNG = 2560
NRG = 4
RPT = RNG // NS
ACC_R = RNG + 8

_mesh = plsc.VectorSubcoreMesh(core_axis_name="c", subcore_axis_name="s")


@functools.partial(
    pl.kernel,
    out_type=jax.ShapeDtypeStruct((NRG, RNG, D), jnp.float32),
    mesh=_mesh,
    scratch_types=[
        pltpu.VMEM((CPC, CHUNK), jnp.int32),
        pltpu.VMEM((CPC, CHUNK), jnp.int32),
        pltpu.VMEM((CPC, CHUNK), jnp.int32),
        pltpu.VMEM((CHUNK, D), jnp.float32),
        pltpu.VMEM((RPT, D), jnp.float32),
        pltpu.VMEM_SHARED((ACC_R, D), jnp.float32),
        pltpu.SemaphoreType.DMA,
    ],
)
def _scatter(g_hbm, pk_hbm, out_hbm, pk_v, src_v, dst_v, rows_v, zbuf, acc, sem):
    c = lax.axis_index("c")
    s = lax.axis_index("s")

    def fill_zero(i, _):
        zbuf[i // 8, pl.ds((i % 8) * 16, 16)] = jnp.zeros((16,), jnp.float32)
        return 0
    lax.fori_loop(0, RPT * 8, fill_zero, 0)

    pltpu.sync_copy(pk_hbm.at[pl.ds(s * CPC, CPC)], pk_v)

    def unpack_src(i, _):
        v = pk_v[i // 8, pl.ds((i % 8) * 16, 16)]
        src_v[i // 8, pl.ds((i % 8) * 16, 16)] = lax.shift_right_logical(v, 14)
        return 0
    lax.fori_loop(0, CPC * 8, unpack_src, 0)

    for p in range(NRG // 2):
        r = c * (NRG // 2) + p
        base = r * RNG

        def unpack_dst(i, _):
            v = pk_v[i // 8, pl.ds((i % 8) * 16, 16)]
            d = (v & 16383) - base
            ok = (d >= 0) & (d < RNG)
            dst_v[i // 8, pl.ds((i % 8) * 16, 16)] = jnp.where(ok, d, RNG)
            return 0
        lax.fori_loop(0, CPC * 8, unpack_dst, 0)

        pltpu.sync_copy(zbuf, acc.at[pl.ds(s * RPT, RPT)])
        plsc.subcore_barrier()

        def body(j, _):
            pltpu.async_copy(g_hbm.at[src_v.at[j]], rows_v, sem).wait()
            pltpu.sync_copy(rows_v, acc.at[dst_v.at[j]], add=True)
            return 0
        lax.fori_loop(0, CPC, body, 0)

        plsc.subcore_barrier()
        pltpu.sync_copy(
            acc.at[pl.ds(s * RPT, RPT)],
            out_hbm.at[r, pl.ds(s * RPT, RPT)],
        )
        plsc.subcore_barrier()


def _dinv_of(degc):
    return jnp.where(degc > 0.0, lax.rsqrt(degc), 0.0)


def _assemble(sp):
    return sp.reshape(NRG * RNG, D)[:N]


def _pack_body(e0_ref, e1_ref, o_ref):
    head = (e0_ref[...] << 14) | e1_ref[...]
    rows = NW * CPW - E // CHUNK
    n = (lax.broadcasted_iota(jnp.int32, (rows, CHUNK), 0) * CHUNK
         + lax.broadcasted_iota(jnp.int32, (rows, CHUNK), 1))
    tail = jnp.where(n < N, n * ((1 << 14) + 1), N)
    o_ref[...] = jnp.concatenate([head, tail], axis=0)


def _degcol_body(dsp_ref, o_ref):
    o_ref[...] = _assemble(dsp_ref[...])[:, 0:1]


def _first_body(x_ref, w_ref, o_ref):
    h = jnp.clip(x_ref[...], -10.0, 10.0)
    o_ref[...] = jnp.dot(h, w_ref[...], preferred_element_type=jnp.float32)


def _scale_body(m_ref, degc_ref, o_ref):
    o_ref[...] = m_ref[...] * _dinv_of(degc_ref[...])


def _mid_body(sp_ref, degc_ref, b_ref, g_ref, bt_ref, w_ref, og_ref, oh_ref):
    dinv = _dinv_of(degc_ref[...])
    t = _assemble(sp_ref[...]) * dinv + b_ref[...]
    mu = jnp.mean(t, axis=0, keepdims=True)
    cen = t - mu
    var = jnp.mean(cen * cen, axis=0, keepdims=True)
    hbn = cen * (g_ref[...] * lax.rsqrt(var + EPS)) + bt_ref[...]
    oh_ref[...] = hbn
    h = jnp.maximum(hbn, 0.0)
    og_ref[...] = jnp.dot(h, w_ref[...], preferred_element_type=jnp.float32) * dinv


_nd = jax.ShapeDtypeStruct((N, D), jnp.float32)
_pack = pl.pallas_call(
    _pack_body, out_shape=jax.ShapeDtypeStruct((NW * CPW, CHUNK), jnp.int32))
_degcol = pl.pallas_call(
    _degcol_body, out_shape=jax.ShapeDtypeStruct((N, 1), jnp.float32))
_first = pl.pallas_call(_first_body, out_shape=_nd)
_scale = pl.pallas_call(_scale_body, out_shape=_nd)
_mid = pl.pallas_call(_mid_body, out_shape=(_nd, _nd))


def kernel(x, edge_index, W1, b1, g1, bt1, W2, b2, g2, bt2, W3, b3, g3, bt3):
    pk2d = _pack(edge_index[0].reshape(E // CHUNK, CHUNK),
                 edge_index[1].reshape(E // CHUNK, CHUNK))

    bs = jnp.stack([b1, b2, b3])[:, None, :]
    gs = jnp.stack([g1, g2, g3])[:, None, :]
    bts = jnp.stack([bt1, bt2, bt3])[:, None, :]
    ws = jnp.stack([W2, W3, jnp.eye(D, dtype=jnp.float32)])

    degc = _degcol(_scatter(jnp.ones((N, D), jnp.float32), pk2d))
    m1 = _first(x, W1)
    g0 = _scale(m1, degc)

    def step(g, params):
        b, gam, bt, w = params
        sp = _scatter(g, pk2d)
        g_next, hbn = _mid(sp, degc, b, gam, bt, w)
        return g_next, hbn

    _, hbns = lax.scan(step, g0, (bs, gs, bts, ws))
    return hbns[-1]
